# baseline (speedup 1.0000x reference)
import functools

import jax
import jax.numpy as jnp
from jax import lax
from jax.experimental import pallas as pl
from jax.experimental.pallas import tpu as pltpu
from jax.experimental.pallas import tpu_sc as plsc

_BATCH = 4


def _make_sc_broadcast(batch, rows, dim, dtype):
    info = plsc.get_sparse_core_info()
    nc, ns = info.num_cores, info.num_subcores
    nw = nc * ns
    rows_per_w = rows // nw
    mesh = plsc.VectorSubcoreMesh(core_axis_name="c", subcore_axis_name="s")

    @functools.partial(
        pl.kernel,
        mesh=mesh,
        out_type=jax.ShapeDtypeStruct((batch, rows, dim), dtype),
        scratch_types=[pltpu.SemaphoreType.DMA],
    )
    def k(w_hbm, out_hbm, sem):
        wid = lax.axis_index("s") * nc + lax.axis_index("c")
        base = wid * rows_per_w
        src = w_hbm.at[pl.ds(base, rows_per_w)]
        copies = [
            pltpu.async_copy(src, out_hbm.at[b, pl.ds(base, rows_per_w)], sem)
            for b in range(batch)
        ]
        for c in copies:
            c.wait()

    return k


def kernel(tokens, W):
    del tokens
    rows, dim = W.shape
    return _make_sc_broadcast(_BATCH, rows, dim, W.dtype)(W)


# speedup vs baseline: 75.8648x; 75.8648x over previous
import jax
import jax.numpy as jnp
from jax.experimental import pallas as pl
from jax.experimental.pallas import tpu as pltpu

_BATCH = 4
_NCHUNK = 8


def _bcast_kernel(w_hbm, out_hbm, vmem, rsem, wsem):
    rows = w_hbm.shape[0]
    cr = rows // _NCHUNK
    reads = []
    for c in range(_NCHUNK):
        rc = pltpu.make_async_copy(
            w_hbm.at[pl.ds(c * cr, cr)], vmem.at[pl.ds(c * cr, cr)], rsem.at[c]
        )
        rc.start()
        reads.append(rc)
    writes = []
    for c in range(_NCHUNK):
        reads[c].wait()
        for b in range(_BATCH):
            wc = pltpu.make_async_copy(
                vmem.at[pl.ds(c * cr, cr)],
                out_hbm.at[b, pl.ds(c * cr, cr)],
                wsem.at[c, b],
            )
            wc.start()
            writes.append(wc)
    for wc in writes:
        wc.wait()


def kernel(tokens, W):
    del tokens
    rows, dim = W.shape
    return pl.pallas_call(
        _bcast_kernel,
        out_shape=jax.ShapeDtypeStruct((_BATCH, rows, dim), W.dtype),
        in_specs=[pl.BlockSpec(memory_space=pl.ANY)],
        out_specs=pl.BlockSpec(memory_space=pl.ANY),
        scratch_shapes=[
            pltpu.VMEM((rows, dim), W.dtype),
            pltpu.SemaphoreType.DMA((_NCHUNK,)),
            pltpu.SemaphoreType.DMA((_NCHUNK, _BATCH)),
        ],
    )(W)
